# baseline (speedup 1.0000x reference)
import functools

import jax
import jax.numpy as jnp
from jax import lax
from jax.experimental import pallas as pl
from jax.experimental.pallas import tpu as pltpu
from jax.experimental.pallas import tpu_sc as plsc

N = 10000
E = 320000
D = 128

NC = 2
NS = 16
L = 16
DH = D // NC

RPT = 640
NPAD = NS * RPT
CH = 2000
NCHK = E // CH
G = 256
FG = DH // L


def _segmax_body(x_hbm, dst_hbm, src_hbm, out_hbm,
                 dst0_v, dst1_v, src0_v, src1_v,
                 pld0_v, pld1_v, psr0_v, psr1_v,
                 rows0_v, rows1_v, agg_v, xs_sh,
                 esem0, esem1, gsem0, gsem1):
    c = lax.axis_index("c")
    s = lax.axis_index("s")
    lo = s * RPT
    iota = lax.iota(jnp.int32, L)
    ninf = jnp.full((L,), -jnp.inf, jnp.float32)

    dst_v = (dst0_v, dst1_v)
    src_v = (src0_v, src1_v)
    pld_v = (pld0_v, pld1_v)
    psr_v = (psr0_v, psr1_v)
    rows_v = (rows0_v, rows1_v)
    esem = (esem0, esem1)
    gsem = (gsem0, gsem1)

    @pl.when(s == 0)
    def _stage():
        pltpu.sync_copy(x_hbm.at[c], xs_sh)
    plsc.subcore_barrier()

    def _init(i, _):
        r = i // FG
        f = i - r * FG
        agg_v[r, pl.ds(f * L, L)] = ninf
        return 0
    lax.fori_loop(0, (RPT + 1) * FG, _init, 0)

    def fire_edges(ci, p):
        pltpu.async_copy(dst_hbm.at[pl.ds(ci * CH, CH)], dst_v[p], esem[p])
        pltpu.async_copy(src_hbm.at[pl.ds(ci * CH, CH)], src_v[p], esem[p])

    def wait_edges(p):
        pltpu.make_async_copy(dst_hbm.at[pl.ds(0, CH)], dst_v[p], esem[p]).wait()
        pltpu.make_async_copy(src_hbm.at[pl.ds(0, CH)], src_v[p], esem[p]).wait()

    def do_filter(p):
        def _filt(i, cnt):
            d = dst_v[p][pl.ds(i * L, L)]
            sv = src_v[p][pl.ds(i * L, L)]
            ld = d - lo
            m = (ld >= 0) & (ld < RPT)
            mi = m.astype(jnp.int32)
            pos = cnt + jnp.cumsum(mi) - 1
            plsc.store_scatter(pld_v[p], [pos], ld, mask=m)
            plsc.store_scatter(psr_v[p], [pos], sv, mask=m)
            return cnt + jnp.sum(mi)
        cnt = lax.fori_loop(0, CH // L, _filt, 0)
        for j in range(G // L):
            tidx = cnt + j * L + iota
            plsc.store_scatter(pld_v[p], [tidx], jnp.full((L,), RPT, jnp.int32))
            plsc.store_scatter(psr_v[p], [tidx], jnp.zeros((L,), jnp.int32))
        return cnt

    H = G // 2

    def fire_gather0(p):
        pltpu.async_copy(xs_sh.at[psr_v[p].at[pl.ds(0, H)]],
                         rows_v[p].at[pl.ds(0, H)], gsem[p])
        pltpu.async_copy(xs_sh.at[psr_v[p].at[pl.ds(H, H)]],
                         rows_v[p].at[pl.ds(H, H)], gsem[p])

    def wait_gather0(p):
        pltpu.make_async_copy(xs_sh.at[psr_v[p].at[pl.ds(0, H)]],
                              rows_v[p].at[pl.ds(0, H)], gsem[p]).wait()
        pltpu.make_async_copy(xs_sh.at[psr_v[p].at[pl.ds(H, H)]],
                              rows_v[p].at[pl.ds(H, H)], gsem[p]).wait()

    def update_block(p, base):
        def _edge(e, _):
            evec = jnp.full((L,), e, jnp.int32)
            dvec = plsc.load_gather(
                pld_v[p], [jnp.full((L,), base, jnp.int32) + evec])
            for f in range(FG):
                col = iota + f * L
                old = plsc.load_gather(agg_v, [dvec, col])
                val = plsc.load_gather(rows_v[p], [evec, col])
                plsc.store_scatter(agg_v, [dvec, col], jnp.maximum(old, val))
            return 0
        lax.fori_loop(0, G, _edge, 0)

    def do_update(p, cnt):
        @pl.when(cnt > 0)
        def _g0():
            update_block(p, 0)
        ngroups = (cnt + G - 1) // G

        def _extra(g, _):
            cp1 = pltpu.async_copy(
                xs_sh.at[psr_v[p].at[pl.ds(g * G, H)]],
                rows_v[p].at[pl.ds(0, H)], gsem[p])
            cp2 = pltpu.async_copy(
                xs_sh.at[psr_v[p].at[pl.ds(g * G + H, H)]],
                rows_v[p].at[pl.ds(H, H)], gsem[p])
            cp1.wait()
            cp2.wait()
            update_block(p, g * G)
            return 0
        lax.fori_loop(1, ngroups, _extra, 0)

    fire_edges(0, 0)
    fire_edges(1, 1)
    wait_edges(0)
    cnt_prev = do_filter(0)
    fire_edges(2, 0)
    fire_gather0(0)

    def _pos(ci, p, cnt_prev, fire_next):
        wait_edges(p)
        cnt_c = do_filter(p)
        if fire_next is not None:
            @pl.when(fire_next < NCHK)
            def _():
                fire_edges(fire_next, p)
        fire_gather0(p)
        wait_gather0(1 - p)
        do_update(1 - p, cnt_prev)
        return cnt_c

    def _pair(j, cnt_prev):
        c1 = 2 * j + 1
        cnt1 = _pos(c1, 1, cnt_prev, c1 + 2)
        cnt2 = _pos(c1 + 1, 0, cnt1, c1 + 3)
        return cnt2
    cnt_prev = lax.fori_loop(0, (NCHK - 2) // 2, _pair, cnt_prev)

    wait_edges(1)
    cnt_last = do_filter(1)
    fire_gather0(1)
    wait_gather0(0)
    do_update(0, cnt_prev)
    wait_gather0(1)
    do_update(1, cnt_last)

    pltpu.sync_copy(agg_v.at[pl.ds(0, RPT)], out_hbm.at[c, pl.ds(lo, RPT)])


_segmax = functools.partial(
    pl.kernel,
    out_type=jax.ShapeDtypeStruct((NC, NPAD, DH), jnp.float32),
    mesh=plsc.VectorSubcoreMesh(core_axis_name="c", subcore_axis_name="s"),
    scratch_types=[
        pltpu.VMEM((CH,), jnp.int32),
        pltpu.VMEM((CH,), jnp.int32),
        pltpu.VMEM((CH,), jnp.int32),
        pltpu.VMEM((CH,), jnp.int32),
        pltpu.VMEM((CH + G,), jnp.int32),
        pltpu.VMEM((CH + G,), jnp.int32),
        pltpu.VMEM((CH + G,), jnp.int32),
        pltpu.VMEM((CH + G,), jnp.int32),
        pltpu.VMEM((G, DH), jnp.float32),
        pltpu.VMEM((G, DH), jnp.float32),
        pltpu.VMEM((RPT + 1, DH), jnp.float32),
        pltpu.VMEM_SHARED((N, DH), jnp.float32),
        pltpu.SemaphoreType.DMA,
        pltpu.SemaphoreType.DMA,
        pltpu.SemaphoreType.DMA,
        pltpu.SemaphoreType.DMA,
    ],
    compiler_params=pltpu.CompilerParams(
        needs_layout_passes=False, use_tc_tiling_on_sc=False),
)(_segmax_body)


def _mm_body(agg_ref, x_ref, wrel_ref, wroot_ref, b_ref, o_ref):
    agg = jnp.concatenate([agg_ref[0], agg_ref[1]], axis=1)
    agg = jnp.where(jnp.isfinite(agg), agg, 0.0)
    x = jnp.concatenate([x_ref[0], x_ref[1]], axis=1)
    h = (
        lax.dot_general(agg, wrel_ref[...], (((1,), (1,)), ((), ())),
                        preferred_element_type=jnp.float32)
        + lax.dot_general(x, wroot_ref[...], (((1,), (1,)), ((), ())),
                          preferred_element_type=jnp.float32)
        + b_ref[...]
    )
    o_ref[0] = h[:, :DH]
    o_ref[1] = h[:, DH:]


def _mm_body_final(agg_ref, x_ref, wrel_ref, wroot_ref, b_ref, o_ref):
    agg = jnp.concatenate([agg_ref[0], agg_ref[1]], axis=1)
    agg = jnp.where(jnp.isfinite(agg), agg, 0.0)
    x = jnp.concatenate([x_ref[0], x_ref[1]], axis=1)
    o_ref[...] = (
        lax.dot_general(agg, wrel_ref[...], (((1,), (1,)), ((), ())),
                        preferred_element_type=jnp.float32)
        + lax.dot_general(x, wroot_ref[...], (((1,), (1,)), ((), ())),
                          preferred_element_type=jnp.float32)
        + b_ref[...]
    )


BR = 1000


def _layer_mm(agg_t, x_t, W_rel, b_rel, W_root, split_out):
    split_spec = pl.BlockSpec((NC, BR, DH), lambda i: (0, i, 0))
    if split_out:
        body, out_shape, out_spec = (
            _mm_body, jax.ShapeDtypeStruct((NC, N, DH), jnp.float32), split_spec)
    else:
        body, out_shape, out_spec = (
            _mm_body_final, jax.ShapeDtypeStruct((N, D), jnp.float32),
            pl.BlockSpec((BR, D), lambda i: (i, 0)))
    return pl.pallas_call(
        body,
        grid=(N // BR,),
        in_specs=[
            split_spec,
            split_spec,
            pl.BlockSpec((D, D), lambda i: (0, 0)),
            pl.BlockSpec((D, D), lambda i: (0, 0)),
            pl.BlockSpec((1, D), lambda i: (0, 0)),
        ],
        out_specs=out_spec,
        out_shape=out_shape,
    )(agg_t, x_t, W_rel, W_root, b_rel.reshape(1, D))


def kernel(x, edge_index, W_rel1, b_rel1, W_root1, W_rel2, b_rel2, W_root2):
    src = edge_index[0]
    dst = edge_index[1]
    x_t = jnp.transpose(x.reshape(N, NC, DH), (1, 0, 2))
    agg1_t = _segmax(x_t, dst, src)[:, :N, :]
    h1_t = _layer_mm(agg1_t, x_t, W_rel1, b_rel1, W_root1, split_out=True)
    agg2_t = _segmax(h1_t, dst, src)[:, :N, :]
    h2 = _layer_mm(agg2_t, h1_t, W_rel2, b_rel2, W_root2, split_out=False)
    return h2


# speedup vs baseline: 1.4137x; 1.4137x over previous
import functools

import jax
import jax.numpy as jnp
from jax import lax
from jax.experimental import pallas as pl
from jax.experimental.pallas import tpu as pltpu
from jax.experimental.pallas import tpu_sc as plsc

N = 10000
E = 320000
D = 128

NC = 2
NS = 16
L = 16
DH = D // NC

RPT = 640
NPAD = NS * RPT
CH = 4000
NCHK = E // CH
G = 320
FG = DH // L
NOV = E // G + NCHK
SB = 16
CBITS = 12


def _update_block(agg_v, pld_ref, rows_v, base, iota):
    def _edge(e, _):
        evec = jnp.full((L,), e, jnp.int32)
        dvec = plsc.load_gather(
            pld_ref, [jnp.full((L,), base, jnp.int32) + evec])
        for f in range(FG):
            col = iota + f * L
            old = plsc.load_gather(agg_v, [dvec, col])
            val = plsc.load_gather(rows_v, [evec, col])
            plsc.store_scatter(agg_v, [dvec, col], jnp.maximum(old, val))
        return 0
    lax.fori_loop(0, G, _edge, 0)


def _init_agg(agg_v):
    ninf = jnp.full((L,), -jnp.inf, jnp.float32)

    def _init(i, _):
        r = i // FG
        f = i - r * FG
        agg_v[r, pl.ds(f * L, L)] = ninf
        return 0
    lax.fori_loop(0, (RPT + 1) * FG, _init, 0)


def _segmax1_body(x_hbm, dst_hbm, src_hbm,
                  out_hbm, pldl_hbm, psrl_hbm, ovp_hbm, ovs_hbm, cnts_hbm,
                  dst_v, src_v, pld_v, psr_v, rows_v, agg_v, xs_sh,
                  sem, wsem):
    c = lax.axis_index("c")
    s = lax.axis_index("s")
    lo = s * RPT
    iota = lax.iota(jnp.int32, L)

    @pl.when(s == 0)
    def _stage():
        pltpu.sync_copy(x_hbm.at[c], xs_sh)
    plsc.subcore_barrier()

    _init_agg(agg_v)

    H = G // 2

    def _chunk(ci, ow):
        @pl.when((c == 0) & (ci > 0))
        def _drain():
            pltpu.make_async_copy(
                pld_v.at[pl.ds(0, G)], pldl_hbm.at[0, pl.ds(0, G)], wsem).wait()
            pltpu.make_async_copy(
                psr_v.at[pl.ds(0, G)], psrl_hbm.at[0, pl.ds(0, G)], wsem).wait()

        pltpu.sync_copy(dst_hbm.at[pl.ds(ci * CH, CH)], dst_v)
        pltpu.sync_copy(src_hbm.at[pl.ds(ci * CH, CH)], src_v)

        def _filt(i, cnt):
            d = dst_v[pl.ds(i * L, L)]
            sv = src_v[pl.ds(i * L, L)]
            ld = d - lo
            m = (ld >= 0) & (ld < RPT)
            mi = m.astype(jnp.int32)
            pos = cnt + jnp.cumsum(mi) - 1
            plsc.store_scatter(pld_v, [pos], ld, mask=m)
            plsc.store_scatter(psr_v, [pos], sv, mask=m)
            return cnt + jnp.sum(mi)
        cnt = lax.fori_loop(0, CH // L, _filt, 0)

        for j in range(G // L):
            tidx = cnt + j * L + iota
            plsc.store_scatter(pld_v, [tidx], jnp.full((L,), RPT, jnp.int32))
            plsc.store_scatter(psr_v, [tidx], jnp.zeros((L,), jnp.int32))

        @pl.when(c == 0)
        def _persist():
            pltpu.async_copy(pld_v.at[pl.ds(0, G)],
                             pldl_hbm.at[s, pl.ds(ci * G, G)], wsem)
            pltpu.async_copy(psr_v.at[pl.ds(0, G)],
                             psrl_hbm.at[s, pl.ds(ci * G, G)], wsem)

        ngroups = (cnt + G - 1) // G

        def _spill(g, owi):
            @pl.when(c == 0)
            def _w():
                pltpu.sync_copy(pld_v.at[pl.ds(g * G, G)],
                                ovp_hbm.at[s, pl.ds(owi * G, G)])
                pltpu.sync_copy(psr_v.at[pl.ds(g * G, G)],
                                ovs_hbm.at[s, pl.ds(owi * G, G)])
            return owi + 1
        ow = lax.fori_loop(1, ngroups, _spill, ow)

        def _group(g, _):
            cp1 = pltpu.async_copy(
                xs_sh.at[psr_v.at[pl.ds(g * G, H)]],
                rows_v.at[pl.ds(0, H)], sem)
            cp2 = pltpu.async_copy(
                xs_sh.at[psr_v.at[pl.ds(g * G + H, H)]],
                rows_v.at[pl.ds(H, H)], sem)
            cp1.wait()
            cp2.wait()
            _update_block(agg_v, pld_v, rows_v, g * G, iota)
            return 0
        lax.fori_loop(0, ngroups, _group, 0)
        return ow
    ow = lax.fori_loop(0, NCHK, _chunk, 0)

    @pl.when(c == 0)
    def _final_drain():
        pltpu.make_async_copy(
            pld_v.at[pl.ds(0, G)], pldl_hbm.at[0, pl.ds(0, G)], wsem).wait()
        pltpu.make_async_copy(
            psr_v.at[pl.ds(0, G)], psrl_hbm.at[0, pl.ds(0, G)], wsem).wait()
        pld_v[pl.ds(0, L)] = jnp.full((L,), ow, jnp.int32)
        pltpu.sync_copy(pld_v.at[pl.ds(0, L)], cnts_hbm.at[s])

    pltpu.sync_copy(agg_v.at[pl.ds(0, RPT)], out_hbm.at[c, pl.ds(lo, RPT)])


_segmax1 = functools.partial(
    pl.kernel,
    out_type=(
        jax.ShapeDtypeStruct((NC, NPAD, DH), jnp.float32),
        jax.ShapeDtypeStruct((NS, NCHK * G), jnp.int32),
        jax.ShapeDtypeStruct((NS, NCHK * G), jnp.int32),
        jax.ShapeDtypeStruct((NS, NOV * G), jnp.int32),
        jax.ShapeDtypeStruct((NS, NOV * G), jnp.int32),
        jax.ShapeDtypeStruct((NS, L), jnp.int32),
    ),
    mesh=plsc.VectorSubcoreMesh(core_axis_name="c", subcore_axis_name="s"),
    scratch_types=[
        pltpu.VMEM((CH,), jnp.int32),
        pltpu.VMEM((CH,), jnp.int32),
        pltpu.VMEM((CH + G,), jnp.int32),
        pltpu.VMEM((CH + G,), jnp.int32),
        pltpu.VMEM((G, DH), jnp.float32),
        pltpu.VMEM((RPT + 1, DH), jnp.float32),
        pltpu.VMEM_SHARED((N, DH), jnp.float32),
        pltpu.SemaphoreType.DMA,
        pltpu.SemaphoreType.DMA,
    ],
    compiler_params=pltpu.CompilerParams(
        needs_layout_passes=False, use_tc_tiling_on_sc=False),
)(_segmax1_body)


def _segmax2_body(x_hbm, pldl_hbm, psrl_hbm, ovp_hbm, ovs_hbm, cnts_hbm,
                  out_hbm, pldc_v, psrc_v, rows_v, agg_v, cnt_v, xs_sh, sem):
    c = lax.axis_index("c")
    s = lax.axis_index("s")
    lo = s * RPT
    iota = lax.iota(jnp.int32, L)

    @pl.when(s == 0)
    def _stage():
        pltpu.sync_copy(x_hbm.at[c], xs_sh)
    plsc.subcore_barrier()

    _init_agg(agg_v)

    H = G // 2

    def _gather_update(list_base):
        cp1 = pltpu.async_copy(
            xs_sh.at[psrc_v.at[pl.ds(list_base, H)]],
            rows_v.at[pl.ds(0, H)], sem)
        cp2 = pltpu.async_copy(
            xs_sh.at[psrc_v.at[pl.ds(list_base + H, H)]],
            rows_v.at[pl.ds(H, H)], sem)
        cp1.wait()
        cp2.wait()
        _update_block(agg_v, pldc_v, rows_v, list_base, iota)

    def _super(sb, _):
        pltpu.sync_copy(pldl_hbm.at[s, pl.ds(sb * SB * G, SB * G)], pldc_v)
        pltpu.sync_copy(psrl_hbm.at[s, pl.ds(sb * SB * G, SB * G)], psrc_v)
        for k in range(SB):
            _gather_update(k * G)
        return 0
    lax.fori_loop(0, NCHK // SB, _super, 0)

    pltpu.sync_copy(cnts_hbm.at[s], cnt_v)
    cv = cnt_v[pl.ds(0, L)]
    now = 0
    for b in range(CBITS):
        bitb = jnp.any(((cv >> b) & 1) == 1)
        now = now + jnp.where(bitb, 1 << b, 0)

    def _ovf(g, _):
        pltpu.sync_copy(ovp_hbm.at[s, pl.ds(g * G, G)],
                        pldc_v.at[pl.ds(0, G)])
        pltpu.sync_copy(ovs_hbm.at[s, pl.ds(g * G, G)],
                        psrc_v.at[pl.ds(0, G)])
        _gather_update(0)
        return 0
    lax.fori_loop(0, now, _ovf, 0)

    pltpu.sync_copy(agg_v.at[pl.ds(0, RPT)], out_hbm.at[c, pl.ds(lo, RPT)])


_segmax2 = functools.partial(
    pl.kernel,
    out_type=jax.ShapeDtypeStruct((NC, NPAD, DH), jnp.float32),
    mesh=plsc.VectorSubcoreMesh(core_axis_name="c", subcore_axis_name="s"),
    scratch_types=[
        pltpu.VMEM((SB * G,), jnp.int32),
        pltpu.VMEM((SB * G,), jnp.int32),
        pltpu.VMEM((G, DH), jnp.float32),
        pltpu.VMEM((RPT + 1, DH), jnp.float32),
        pltpu.VMEM((L,), jnp.int32),
        pltpu.VMEM_SHARED((N, DH), jnp.float32),
        pltpu.SemaphoreType.DMA,
    ],
    compiler_params=pltpu.CompilerParams(
        needs_layout_passes=False, use_tc_tiling_on_sc=False),
)(_segmax2_body)


def _mm_body(agg_ref, x_ref, wrel_ref, wroot_ref, b_ref, o_ref):
    agg = jnp.concatenate([agg_ref[0], agg_ref[1]], axis=1)
    agg = jnp.where(jnp.isfinite(agg), agg, 0.0)
    x = jnp.concatenate([x_ref[0], x_ref[1]], axis=1)
    h = (
        lax.dot_general(agg, wrel_ref[...], (((1,), (1,)), ((), ())),
                        preferred_element_type=jnp.float32)
        + lax.dot_general(x, wroot_ref[...], (((1,), (1,)), ((), ())),
                          preferred_element_type=jnp.float32)
        + b_ref[...]
    )
    o_ref[0] = h[:, :DH]
    o_ref[1] = h[:, DH:]


def _mm_body_final(agg_ref, x_ref, wrel_ref, wroot_ref, b_ref, o_ref):
    agg = jnp.concatenate([agg_ref[0], agg_ref[1]], axis=1)
    agg = jnp.where(jnp.isfinite(agg), agg, 0.0)
    x = jnp.concatenate([x_ref[0], x_ref[1]], axis=1)
    o_ref[...] = (
        lax.dot_general(agg, wrel_ref[...], (((1,), (1,)), ((), ())),
                        preferred_element_type=jnp.float32)
        + lax.dot_general(x, wroot_ref[...], (((1,), (1,)), ((), ())),
                          preferred_element_type=jnp.float32)
        + b_ref[...]
    )


BR = 1000


def _layer_mm(agg_t, x_t, W_rel, b_rel, W_root, split_out):
    split_spec = pl.BlockSpec((NC, BR, DH), lambda i: (0, i, 0))
    if split_out:
        body, out_shape, out_spec = (
            _mm_body, jax.ShapeDtypeStruct((NC, N, DH), jnp.float32), split_spec)
    else:
        body, out_shape, out_spec = (
            _mm_body_final, jax.ShapeDtypeStruct((N, D), jnp.float32),
            pl.BlockSpec((BR, D), lambda i: (i, 0)))
    return pl.pallas_call(
        body,
        grid=(N // BR,),
        in_specs=[
            split_spec,
            split_spec,
            pl.BlockSpec((D, D), lambda i: (0, 0)),
            pl.BlockSpec((D, D), lambda i: (0, 0)),
            pl.BlockSpec((1, D), lambda i: (0, 0)),
        ],
        out_specs=out_spec,
        out_shape=out_shape,
    )(agg_t, x_t, W_rel, W_root, b_rel.reshape(1, D))


def kernel(x, edge_index, W_rel1, b_rel1, W_root1, W_rel2, b_rel2, W_root2):
    src = edge_index[0]
    dst = edge_index[1]
    x_t = jnp.transpose(x.reshape(N, NC, DH), (1, 0, 2))
    agg1_t, pldl, psrl, ovp, ovs, cnts = _segmax1(x_t, dst, src)
    h1_t = _layer_mm(agg1_t[:, :N, :], x_t, W_rel1, b_rel1, W_root1,
                     split_out=True)
    agg2_t = _segmax2(h1_t, pldl, psrl, ovp, ovs, cnts)
    h2 = _layer_mm(agg2_t[:, :N, :], h1_t, W_rel2, b_rel2, W_root2,
                   split_out=False)
    return h2


# speedup vs baseline: 1.4606x; 1.0332x over previous
import functools

import jax
import jax.numpy as jnp
from jax import lax
from jax.experimental import pallas as pl
from jax.experimental.pallas import tpu as pltpu
from jax.experimental.pallas import tpu_sc as plsc

N = 10000
E = 320000
D = 128

NC = 2
NS = 16
L = 16
DH = D // NC

RPT = 640
NPAD = NS * RPT
CH = 4000
NCHK = E // CH
G = 320
FG = DH // L
NOV = E // G + NCHK
SB = 16
CBITS = 12


def _update_block(agg_v, pld_ref, rows_v, base, iota):
    def _edge(e, _):
        evec = jnp.full((L,), e, jnp.int32)
        dvec = plsc.load_gather(
            pld_ref, [jnp.full((L,), base, jnp.int32) + evec])
        for f in range(FG):
            col = iota + f * L
            old = plsc.load_gather(agg_v, [dvec, col])
            val = plsc.load_gather(rows_v, [evec, col])
            plsc.store_scatter(agg_v, [dvec, col], jnp.maximum(old, val))
        return 0
    lax.fori_loop(0, G, _edge, 0)


def _init_agg(agg_v):
    ninf = jnp.full((L,), -jnp.inf, jnp.float32)

    def _init(i, _):
        r = i // FG
        f = i - r * FG
        agg_v[r, pl.ds(f * L, L)] = ninf
        return 0
    lax.fori_loop(0, (RPT + 1) * FG, _init, 0)


def _segmax1_body(x_hbm, dst_hbm, src_hbm,
                  out_hbm, pldl_hbm, psrl_hbm, ovp_hbm, ovs_hbm, cnts_hbm,
                  dst_v, src_v, pld_v, psr_v, rows_v, agg_v, xs_sh,
                  sem, wsem):
    c = lax.axis_index("c")
    s = lax.axis_index("s")
    lo = s * RPT
    iota = lax.iota(jnp.int32, L)

    @pl.when(s == 0)
    def _stage():
        pltpu.sync_copy(x_hbm.at[c], xs_sh)
    plsc.subcore_barrier()

    _init_agg(agg_v)

    H = G // 2

    def _chunk(ci, ow):
        @pl.when((c == 0) & (ci > 0))
        def _drain():
            pltpu.make_async_copy(
                pld_v.at[pl.ds(0, G)], pldl_hbm.at[0, pl.ds(0, G)], wsem).wait()
            pltpu.make_async_copy(
                psr_v.at[pl.ds(0, G)], psrl_hbm.at[0, pl.ds(0, G)], wsem).wait()

        pltpu.sync_copy(dst_hbm.at[pl.ds(ci * CH, CH)], dst_v)
        pltpu.sync_copy(src_hbm.at[pl.ds(ci * CH, CH)], src_v)

        def _filt(i, cnt):
            d = dst_v[pl.ds(i * L, L)]
            sv = src_v[pl.ds(i * L, L)]
            ld = d - lo
            m = (ld >= 0) & (ld < RPT)
            mi = m.astype(jnp.int32)
            plsc.store_compressed(pld_v.at[pl.ds(cnt, L)], ld, mask=m)
            plsc.store_compressed(psr_v.at[pl.ds(cnt, L)], sv, mask=m)
            return cnt + jnp.sum(mi)
        cnt = lax.fori_loop(0, CH // L, _filt, 0)

        for j in range(G // L):
            tidx = cnt + j * L + iota
            plsc.store_scatter(pld_v, [tidx], jnp.full((L,), RPT, jnp.int32))
            plsc.store_scatter(psr_v, [tidx], jnp.zeros((L,), jnp.int32))

        @pl.when(c == 0)
        def _persist():
            pltpu.async_copy(pld_v.at[pl.ds(0, G)],
                             pldl_hbm.at[s, pl.ds(ci * G, G)], wsem)
            pltpu.async_copy(psr_v.at[pl.ds(0, G)],
                             psrl_hbm.at[s, pl.ds(ci * G, G)], wsem)

        ngroups = (cnt + G - 1) // G

        def _spill(g, owi):
            @pl.when(c == 0)
            def _w():
                pltpu.sync_copy(pld_v.at[pl.ds(g * G, G)],
                                ovp_hbm.at[s, pl.ds(owi * G, G)])
                pltpu.sync_copy(psr_v.at[pl.ds(g * G, G)],
                                ovs_hbm.at[s, pl.ds(owi * G, G)])
            return owi + 1
        ow = lax.fori_loop(1, ngroups, _spill, ow)

        def _group(g, _):
            cp1 = pltpu.async_copy(
                xs_sh.at[psr_v.at[pl.ds(g * G, H)]],
                rows_v.at[pl.ds(0, H)], sem)
            cp2 = pltpu.async_copy(
                xs_sh.at[psr_v.at[pl.ds(g * G + H, H)]],
                rows_v.at[pl.ds(H, H)], sem)
            cp1.wait()
            cp2.wait()
            _update_block(agg_v, pld_v, rows_v, g * G, iota)
            return 0
        lax.fori_loop(0, ngroups, _group, 0)
        return ow
    ow = lax.fori_loop(0, NCHK, _chunk, 0)

    @pl.when(c == 0)
    def _final_drain():
        pltpu.make_async_copy(
            pld_v.at[pl.ds(0, G)], pldl_hbm.at[0, pl.ds(0, G)], wsem).wait()
        pltpu.make_async_copy(
            psr_v.at[pl.ds(0, G)], psrl_hbm.at[0, pl.ds(0, G)], wsem).wait()
        pld_v[pl.ds(0, L)] = jnp.full((L,), ow, jnp.int32)
        pltpu.sync_copy(pld_v.at[pl.ds(0, L)], cnts_hbm.at[s])

    pltpu.sync_copy(agg_v.at[pl.ds(0, RPT)], out_hbm.at[c, pl.ds(lo, RPT)])


_segmax1 = functools.partial(
    pl.kernel,
    out_type=(
        jax.ShapeDtypeStruct((NC, NPAD, DH), jnp.float32),
        jax.ShapeDtypeStruct((NS, NCHK * G), jnp.int32),
        jax.ShapeDtypeStruct((NS, NCHK * G), jnp.int32),
        jax.ShapeDtypeStruct((NS, NOV * G), jnp.int32),
        jax.ShapeDtypeStruct((NS, NOV * G), jnp.int32),
        jax.ShapeDtypeStruct((NS, L), jnp.int32),
    ),
    mesh=plsc.VectorSubcoreMesh(core_axis_name="c", subcore_axis_name="s"),
    scratch_types=[
        pltpu.VMEM((CH,), jnp.int32),
        pltpu.VMEM((CH,), jnp.int32),
        pltpu.VMEM((CH + G,), jnp.int32),
        pltpu.VMEM((CH + G,), jnp.int32),
        pltpu.VMEM((G, DH), jnp.float32),
        pltpu.VMEM((RPT + 1, DH), jnp.float32),
        pltpu.VMEM_SHARED((N, DH), jnp.float32),
        pltpu.SemaphoreType.DMA,
        pltpu.SemaphoreType.DMA,
    ],
    compiler_params=pltpu.CompilerParams(
        needs_layout_passes=False, use_tc_tiling_on_sc=False),
)(_segmax1_body)


def _segmax2_body(x_hbm, pldl_hbm, psrl_hbm, ovp_hbm, ovs_hbm, cnts_hbm,
                  out_hbm, pldc_v, psrc_v, rows_v, agg_v, cnt_v, xs_sh, sem):
    c = lax.axis_index("c")
    s = lax.axis_index("s")
    lo = s * RPT
    iota = lax.iota(jnp.int32, L)

    @pl.when(s == 0)
    def _stage():
        pltpu.sync_copy(x_hbm.at[c], xs_sh)
    plsc.subcore_barrier()

    _init_agg(agg_v)

    H = G // 2

    def _gather_update(list_base):
        cp1 = pltpu.async_copy(
            xs_sh.at[psrc_v.at[pl.ds(list_base, H)]],
            rows_v.at[pl.ds(0, H)], sem)
        cp2 = pltpu.async_copy(
            xs_sh.at[psrc_v.at[pl.ds(list_base + H, H)]],
            rows_v.at[pl.ds(H, H)], sem)
        cp1.wait()
        cp2.wait()
        _update_block(agg_v, pldc_v, rows_v, list_base, iota)

    def _super(sb, _):
        pltpu.sync_copy(pldl_hbm.at[s, pl.ds(sb * SB * G, SB * G)], pldc_v)
        pltpu.sync_copy(psrl_hbm.at[s, pl.ds(sb * SB * G, SB * G)], psrc_v)
        for k in range(SB):
            _gather_update(k * G)
        return 0
    lax.fori_loop(0, NCHK // SB, _super, 0)

    pltpu.sync_copy(cnts_hbm.at[s], cnt_v)
    cv = cnt_v[pl.ds(0, L)]
    now = 0
    for b in range(CBITS):
        bitb = jnp.any(((cv >> b) & 1) == 1)
        now = now + jnp.where(bitb, 1 << b, 0)

    def _ovf(g, _):
        pltpu.sync_copy(ovp_hbm.at[s, pl.ds(g * G, G)],
                        pldc_v.at[pl.ds(0, G)])
        pltpu.sync_copy(ovs_hbm.at[s, pl.ds(g * G, G)],
                        psrc_v.at[pl.ds(0, G)])
        _gather_update(0)
        return 0
    lax.fori_loop(0, now, _ovf, 0)

    pltpu.sync_copy(agg_v.at[pl.ds(0, RPT)], out_hbm.at[c, pl.ds(lo, RPT)])


_segmax2 = functools.partial(
    pl.kernel,
    out_type=jax.ShapeDtypeStruct((NC, NPAD, DH), jnp.float32),
    mesh=plsc.VectorSubcoreMesh(core_axis_name="c", subcore_axis_name="s"),
    scratch_types=[
        pltpu.VMEM((SB * G,), jnp.int32),
        pltpu.VMEM((SB * G,), jnp.int32),
        pltpu.VMEM((G, DH), jnp.float32),
        pltpu.VMEM((RPT + 1, DH), jnp.float32),
        pltpu.VMEM((L,), jnp.int32),
        pltpu.VMEM_SHARED((N, DH), jnp.float32),
        pltpu.SemaphoreType.DMA,
    ],
    compiler_params=pltpu.CompilerParams(
        needs_layout_passes=False, use_tc_tiling_on_sc=False),
)(_segmax2_body)


def _mm_body(agg_ref, x_ref, wrel_ref, wroot_ref, b_ref, o_ref):
    agg = jnp.concatenate([agg_ref[0], agg_ref[1]], axis=1)
    agg = jnp.where(jnp.isfinite(agg), agg, 0.0)
    x = jnp.concatenate([x_ref[0], x_ref[1]], axis=1)
    h = (
        lax.dot_general(agg, wrel_ref[...], (((1,), (1,)), ((), ())),
                        preferred_element_type=jnp.float32)
        + lax.dot_general(x, wroot_ref[...], (((1,), (1,)), ((), ())),
                          preferred_element_type=jnp.float32)
        + b_ref[...]
    )
    o_ref[0] = h[:, :DH]
    o_ref[1] = h[:, DH:]


def _mm_body_final(agg_ref, x_ref, wrel_ref, wroot_ref, b_ref, o_ref):
    agg = jnp.concatenate([agg_ref[0], agg_ref[1]], axis=1)
    agg = jnp.where(jnp.isfinite(agg), agg, 0.0)
    x = jnp.concatenate([x_ref[0], x_ref[1]], axis=1)
    o_ref[...] = (
        lax.dot_general(agg, wrel_ref[...], (((1,), (1,)), ((), ())),
                        preferred_element_type=jnp.float32)
        + lax.dot_general(x, wroot_ref[...], (((1,), (1,)), ((), ())),
                          preferred_element_type=jnp.float32)
        + b_ref[...]
    )


BR = 1000


def _layer_mm(agg_t, x_t, W_rel, b_rel, W_root, split_out):
    split_spec = pl.BlockSpec((NC, BR, DH), lambda i: (0, i, 0))
    if split_out:
        body, out_shape, out_spec = (
            _mm_body, jax.ShapeDtypeStruct((NC, N, DH), jnp.float32), split_spec)
    else:
        body, out_shape, out_spec = (
            _mm_body_final, jax.ShapeDtypeStruct((N, D), jnp.float32),
            pl.BlockSpec((BR, D), lambda i: (i, 0)))
    return pl.pallas_call(
        body,
        grid=(N // BR,),
        in_specs=[
            split_spec,
            split_spec,
            pl.BlockSpec((D, D), lambda i: (0, 0)),
            pl.BlockSpec((D, D), lambda i: (0, 0)),
            pl.BlockSpec((1, D), lambda i: (0, 0)),
        ],
        out_specs=out_spec,
        out_shape=out_shape,
    )(agg_t, x_t, W_rel, W_root, b_rel.reshape(1, D))


def kernel(x, edge_index, W_rel1, b_rel1, W_root1, W_rel2, b_rel2, W_root2):
    src = edge_index[0]
    dst = edge_index[1]
    x_t = jnp.transpose(x.reshape(N, NC, DH), (1, 0, 2))
    agg1_t, pldl, psrl, ovp, ovs, cnts = _segmax1(x_t, dst, src)
    h1_t = _layer_mm(agg1_t[:, :N, :], x_t, W_rel1, b_rel1, W_root1,
                     split_out=True)
    agg2_t = _segmax2(h1_t, pldl, psrl, ovp, ovs, cnts)
    h2 = _layer_mm(agg2_t[:, :N, :], h1_t, W_rel2, b_rel2, W_root2,
                   split_out=False)
    return h2


# speedup vs baseline: 1.5243x; 1.0436x over previous
import functools

import jax
import jax.numpy as jnp
from jax import lax
from jax.experimental import pallas as pl
from jax.experimental.pallas import tpu as pltpu
from jax.experimental.pallas import tpu_sc as plsc

N = 10000
E = 320000
D = 128

NC = 2
NS = 16
L = 16
DH = D // NC

RPT = 640
NPAD = NS * RPT
CH = 5000
NCHK = E // CH
G = 384
FG = DH // L
NOV = E // G + NCHK
SB = 16
CBITS = 12


def _update_block(agg_v, pld_ref, rows_v, base, iota):
    def _edge(e, _):
        evec = jnp.full((L,), e, jnp.int32)
        dvec = plsc.load_gather(
            pld_ref, [jnp.full((L,), base, jnp.int32) + evec])
        for f in range(FG):
            col = iota + f * L
            old = plsc.load_gather(agg_v, [dvec, col])
            val = plsc.load_gather(rows_v, [evec, col])
            plsc.store_scatter(agg_v, [dvec, col], jnp.maximum(old, val))
        return 0
    lax.fori_loop(0, G, _edge, 0)


def _init_agg(agg_v):
    ninf = jnp.full((L,), -jnp.inf, jnp.float32)

    def _init(i, _):
        r = i // FG
        f = i - r * FG
        agg_v[r, pl.ds(f * L, L)] = ninf
        return 0
    lax.fori_loop(0, (RPT + 1) * FG, _init, 0)


def _segmax1_body(x_hbm, dst_hbm, src_hbm,
                  out_hbm, pldl_hbm, psrl_hbm, ovp_hbm, ovs_hbm, cnts_hbm,
                  dst_v, src_v, pld_v, psr_v, rows_v, agg_v, xs_sh,
                  sem, wsem):
    c = lax.axis_index("c")
    s = lax.axis_index("s")
    lo = s * RPT
    iota = lax.iota(jnp.int32, L)

    @pl.when(s == 0)
    def _stage():
        pltpu.sync_copy(x_hbm.at[c], xs_sh)
    plsc.subcore_barrier()

    _init_agg(agg_v)

    H = G // 2

    def _chunk(ci, ow):
        @pl.when((c == 0) & (ci > 0))
        def _drain():
            pltpu.make_async_copy(
                pld_v.at[pl.ds(0, G)], pldl_hbm.at[0, pl.ds(0, G)], wsem).wait()
            pltpu.make_async_copy(
                psr_v.at[pl.ds(0, G)], psrl_hbm.at[0, pl.ds(0, G)], wsem).wait()

        pltpu.sync_copy(dst_hbm.at[pl.ds(ci * CH, CH)], dst_v)
        pltpu.sync_copy(src_hbm.at[pl.ds(ci * CH, CH)], src_v)

        def _filt(i, cnt):
            d = dst_v[pl.ds(i * L, L)]
            sv = src_v[pl.ds(i * L, L)]
            ld = d - lo
            m = (ld >= 0) & (ld < RPT)
            mi = m.astype(jnp.int32)
            plsc.store_compressed(pld_v.at[pl.ds(cnt, L)], ld, mask=m)
            plsc.store_compressed(psr_v.at[pl.ds(cnt, L)], sv, mask=m)
            return cnt + jnp.sum(mi)
        cnt = lax.fori_loop(0, CH // L, _filt, 0)

        for j in range(G // L):
            tidx = cnt + j * L + iota
            plsc.store_scatter(pld_v, [tidx], jnp.full((L,), RPT, jnp.int32))
            plsc.store_scatter(psr_v, [tidx], jnp.zeros((L,), jnp.int32))

        @pl.when(c == 0)
        def _persist():
            pltpu.async_copy(pld_v.at[pl.ds(0, G)],
                             pldl_hbm.at[s, pl.ds(ci * G, G)], wsem)
            pltpu.async_copy(psr_v.at[pl.ds(0, G)],
                             psrl_hbm.at[s, pl.ds(ci * G, G)], wsem)

        ngroups = (cnt + G - 1) // G

        def _spill(g, owi):
            @pl.when(c == 0)
            def _w():
                pltpu.sync_copy(pld_v.at[pl.ds(g * G, G)],
                                ovp_hbm.at[s, pl.ds(owi * G, G)])
                pltpu.sync_copy(psr_v.at[pl.ds(g * G, G)],
                                ovs_hbm.at[s, pl.ds(owi * G, G)])
            return owi + 1
        ow = lax.fori_loop(1, ngroups, _spill, ow)

        def _group(g, _):
            cp1 = pltpu.async_copy(
                xs_sh.at[psr_v.at[pl.ds(g * G, H)]],
                rows_v.at[pl.ds(0, H)], sem)
            cp2 = pltpu.async_copy(
                xs_sh.at[psr_v.at[pl.ds(g * G + H, H)]],
                rows_v.at[pl.ds(H, H)], sem)
            cp1.wait()
            cp2.wait()
            _update_block(agg_v, pld_v, rows_v, g * G, iota)
            return 0
        lax.fori_loop(0, ngroups, _group, 0)
        return ow
    ow = lax.fori_loop(0, NCHK, _chunk, 0)

    @pl.when(c == 0)
    def _final_drain():
        pltpu.make_async_copy(
            pld_v.at[pl.ds(0, G)], pldl_hbm.at[0, pl.ds(0, G)], wsem).wait()
        pltpu.make_async_copy(
            psr_v.at[pl.ds(0, G)], psrl_hbm.at[0, pl.ds(0, G)], wsem).wait()
        pld_v[pl.ds(0, L)] = jnp.full((L,), ow, jnp.int32)
        pltpu.sync_copy(pld_v.at[pl.ds(0, L)], cnts_hbm.at[s])

    pltpu.sync_copy(agg_v.at[pl.ds(0, RPT)], out_hbm.at[c, pl.ds(lo, RPT)])


_segmax1 = functools.partial(
    pl.kernel,
    out_type=(
        jax.ShapeDtypeStruct((NC, NPAD, DH), jnp.float32),
        jax.ShapeDtypeStruct((NS, NCHK * G), jnp.int32),
        jax.ShapeDtypeStruct((NS, NCHK * G), jnp.int32),
        jax.ShapeDtypeStruct((NS, NOV * G), jnp.int32),
        jax.ShapeDtypeStruct((NS, NOV * G), jnp.int32),
        jax.ShapeDtypeStruct((NS, L), jnp.int32),
    ),
    mesh=plsc.VectorSubcoreMesh(core_axis_name="c", subcore_axis_name="s"),
    scratch_types=[
        pltpu.VMEM((CH,), jnp.int32),
        pltpu.VMEM((CH,), jnp.int32),
        pltpu.VMEM((CH + G,), jnp.int32),
        pltpu.VMEM((CH + G,), jnp.int32),
        pltpu.VMEM((G, DH), jnp.float32),
        pltpu.VMEM((RPT + 1, DH), jnp.float32),
        pltpu.VMEM_SHARED((N, DH), jnp.float32),
        pltpu.SemaphoreType.DMA,
        pltpu.SemaphoreType.DMA,
    ],
    compiler_params=pltpu.CompilerParams(
        needs_layout_passes=False, use_tc_tiling_on_sc=False),
)(_segmax1_body)


def _segmax2_body(x_hbm, pldl_hbm, psrl_hbm, ovp_hbm, ovs_hbm, cnts_hbm,
                  out_hbm, pldc_v, psrc_v, rows_v, agg_v, cnt_v, xs_sh, sem):
    c = lax.axis_index("c")
    s = lax.axis_index("s")
    lo = s * RPT
    iota = lax.iota(jnp.int32, L)

    @pl.when(s == 0)
    def _stage():
        pltpu.sync_copy(x_hbm.at[c], xs_sh)
    plsc.subcore_barrier()

    _init_agg(agg_v)

    H = G // 2

    def _gather_update(list_base):
        cp1 = pltpu.async_copy(
            xs_sh.at[psrc_v.at[pl.ds(list_base, H)]],
            rows_v.at[pl.ds(0, H)], sem)
        cp2 = pltpu.async_copy(
            xs_sh.at[psrc_v.at[pl.ds(list_base + H, H)]],
            rows_v.at[pl.ds(H, H)], sem)
        cp1.wait()
        cp2.wait()
        _update_block(agg_v, pldc_v, rows_v, list_base, iota)

    def _super(sb, _):
        pltpu.sync_copy(pldl_hbm.at[s, pl.ds(sb * SB * G, SB * G)], pldc_v)
        pltpu.sync_copy(psrl_hbm.at[s, pl.ds(sb * SB * G, SB * G)], psrc_v)
        for k in range(SB):
            _gather_update(k * G)
        return 0
    lax.fori_loop(0, NCHK // SB, _super, 0)

    pltpu.sync_copy(cnts_hbm.at[s], cnt_v)
    cv = cnt_v[pl.ds(0, L)]
    now = 0
    for b in range(CBITS):
        bitb = jnp.any(((cv >> b) & 1) == 1)
        now = now + jnp.where(bitb, 1 << b, 0)

    def _ovf(g, _):
        pltpu.sync_copy(ovp_hbm.at[s, pl.ds(g * G, G)],
                        pldc_v.at[pl.ds(0, G)])
        pltpu.sync_copy(ovs_hbm.at[s, pl.ds(g * G, G)],
                        psrc_v.at[pl.ds(0, G)])
        _gather_update(0)
        return 0
    lax.fori_loop(0, now, _ovf, 0)

    pltpu.sync_copy(agg_v.at[pl.ds(0, RPT)], out_hbm.at[c, pl.ds(lo, RPT)])


_segmax2 = functools.partial(
    pl.kernel,
    out_type=jax.ShapeDtypeStruct((NC, NPAD, DH), jnp.float32),
    mesh=plsc.VectorSubcoreMesh(core_axis_name="c", subcore_axis_name="s"),
    scratch_types=[
        pltpu.VMEM((SB * G,), jnp.int32),
        pltpu.VMEM((SB * G,), jnp.int32),
        pltpu.VMEM((G, DH), jnp.float32),
        pltpu.VMEM((RPT + 1, DH), jnp.float32),
        pltpu.VMEM((L,), jnp.int32),
        pltpu.VMEM_SHARED((N, DH), jnp.float32),
        pltpu.SemaphoreType.DMA,
    ],
    compiler_params=pltpu.CompilerParams(
        needs_layout_passes=False, use_tc_tiling_on_sc=False),
)(_segmax2_body)


def _mm_body(agg_ref, x_ref, wrel_ref, wroot_ref, b_ref, o_ref):
    agg = jnp.concatenate([agg_ref[0], agg_ref[1]], axis=1)
    agg = jnp.where(jnp.isfinite(agg), agg, 0.0)
    x = jnp.concatenate([x_ref[0], x_ref[1]], axis=1)
    h = (
        lax.dot_general(agg, wrel_ref[...], (((1,), (1,)), ((), ())),
                        preferred_element_type=jnp.float32)
        + lax.dot_general(x, wroot_ref[...], (((1,), (1,)), ((), ())),
                          preferred_element_type=jnp.float32)
        + b_ref[...]
    )
    o_ref[0] = h[:, :DH]
    o_ref[1] = h[:, DH:]


def _mm_body_final(agg_ref, x_ref, wrel_ref, wroot_ref, b_ref, o_ref):
    agg = jnp.concatenate([agg_ref[0], agg_ref[1]], axis=1)
    agg = jnp.where(jnp.isfinite(agg), agg, 0.0)
    x = jnp.concatenate([x_ref[0], x_ref[1]], axis=1)
    o_ref[...] = (
        lax.dot_general(agg, wrel_ref[...], (((1,), (1,)), ((), ())),
                        preferred_element_type=jnp.float32)
        + lax.dot_general(x, wroot_ref[...], (((1,), (1,)), ((), ())),
                          preferred_element_type=jnp.float32)
        + b_ref[...]
    )


BR = 1000


def _layer_mm(agg_t, x_t, W_rel, b_rel, W_root, split_out):
    split_spec = pl.BlockSpec((NC, BR, DH), lambda i: (0, i, 0))
    if split_out:
        body, out_shape, out_spec = (
            _mm_body, jax.ShapeDtypeStruct((NC, N, DH), jnp.float32), split_spec)
    else:
        body, out_shape, out_spec = (
            _mm_body_final, jax.ShapeDtypeStruct((N, D), jnp.float32),
            pl.BlockSpec((BR, D), lambda i: (i, 0)))
    return pl.pallas_call(
        body,
        grid=(N // BR,),
        in_specs=[
            split_spec,
            split_spec,
            pl.BlockSpec((D, D), lambda i: (0, 0)),
            pl.BlockSpec((D, D), lambda i: (0, 0)),
            pl.BlockSpec((1, D), lambda i: (0, 0)),
        ],
        out_specs=out_spec,
        out_shape=out_shape,
    )(agg_t, x_t, W_rel, W_root, b_rel.reshape(1, D))


def kernel(x, edge_index, W_rel1, b_rel1, W_root1, W_rel2, b_rel2, W_root2):
    src = edge_index[0]
    dst = edge_index[1]
    x_t = jnp.transpose(x.reshape(N, NC, DH), (1, 0, 2))
    agg1_t, pldl, psrl, ovp, ovs, cnts = _segmax1(x_t, dst, src)
    h1_t = _layer_mm(agg1_t[:, :N, :], x_t, W_rel1, b_rel1, W_root1,
                     split_out=True)
    agg2_t = _segmax2(h1_t, pldl, psrl, ovp, ovs, cnts)
    h2 = _layer_mm(agg2_t[:, :N, :], h1_t, W_rel2, b_rel2, W_root2,
                   split_out=False)
    return h2
